# Initial kernel scaffold; baseline (speedup 1.0000x reference)
import jax
import jax.numpy as jnp
from jax.experimental import pallas as pl


def kernel(x, edge_index, W1, b1, W2, b2, W3, b3):
    raise NotImplementedError("write your pallas kernel here")


# speedup vs baseline: 5.1371x; 5.1371x over previous
import functools

import jax
import jax.numpy as jnp
from jax import lax
from jax.experimental import pallas as pl
from jax.experimental.pallas import tpu as pltpu
from jax.experimental.pallas import tpu_sc as plsc

D = 128
NC, NS, L = 2, 16, 16
NW = NC * NS
CHUNK = 128
G = 80
HALF = 40
EPAD = NW * G * CHUNK
GD = EPAD // NS // CHUNK
NPAD = 10240
RPS = NPAD // NS
BLK = 512


def _mesh():
    return plsc.VectorSubcoreMesh(
        core_axis_name="c", subcore_axis_name="s",
        num_cores=NC, num_subcores=NS)


def _fill2d(ref, nrows, ncols, val):
    def rbody(i, _):
        def cbody(j, _):
            ref[i, pl.ds(j * L, L)] = jnp.full((L,), val, jnp.float32)
            return 0
        return lax.fori_loop(0, ncols // L, cbody, 0)
    lax.fori_loop(0, nrows, rbody, 0)


def _sc_degree(e2):
    @functools.partial(
        pl.kernel,
        out_type=jax.ShapeDtypeStruct((NC, NPAD, D), jnp.float32),
        mesh=_mesh(),
        scratch_types=[
            pltpu.VMEM((GD // 2,), jnp.int32),
            pltpu.VMEM((GD, CHUNK), jnp.int32),
            pltpu.VMEM((CHUNK, D), jnp.float32),
            pltpu.VMEM_SHARED((NPAD, D), jnp.float32),
            pltpu.SemaphoreType.DMA,
        ],
    )
    def k(e_hbm, out_hbm, cidx_v, idx_v, ones_v, acc_sh, semi):
        c = lax.axis_index("c")
        s = lax.axis_index("s")
        _fill2d(ones_v, CHUNK, D, 0.0)
        for j in range(RPS // CHUNK):
            pltpu.sync_copy(ones_v,
                            acc_sh.at[pl.ds(s * RPS + j * CHUNK, CHUNK)])
        hh = GD // 2
        base = c * NW * G + s * GD
        for h in range(2):
            def cb(j, _):
                cidx_v[pl.ds(j * L, L)] = lax.iota(jnp.int32, L) + (
                    base + h * hh + j * L)
                return 0
            lax.fori_loop(0, hh // L, cb, 0)
            pltpu.async_copy(e_hbm.at[cidx_v],
                             idx_v.at[pl.ds(h * hh, hh)], semi).wait()
        _fill2d(ones_v, CHUNK, D, 1.0)
        plsc.subcore_barrier()

        def body(g, _):
            pltpu.sync_copy(ones_v, acc_sh.at[idx_v.at[g]], add=True)
            return 0
        lax.fori_loop(0, GD, body, 0)
        plsc.subcore_barrier()
        for j in range(RPS // CHUNK):
            off = s * RPS + j * CHUNK
            pltpu.sync_copy(acc_sh.at[pl.ds(off, CHUNK)], ones_v)
            pltpu.sync_copy(ones_v, out_hbm.at[c].at[pl.ds(off, CHUNK)])

    return k(e2)


def _sc_edgepass(table, e2):
    nbuf = 2

    @functools.partial(
        pl.kernel,
        out_type=jax.ShapeDtypeStruct((NC, NPAD, D), jnp.float32),
        mesh=_mesh(),
        scratch_types=[
            pltpu.VMEM((2 * HALF,), jnp.int32),
            pltpu.VMEM((2 * HALF, CHUNK), jnp.int32),
            pltpu.VMEM((nbuf, CHUNK, D), jnp.float32),
            pltpu.VMEM_SHARED((NPAD, D), jnp.float32),
            pltpu.SemaphoreType.DMA((nbuf,)),
            pltpu.SemaphoreType.DMA,
        ],
    )
    def k(t_hbm, e_hbm, out_hbm, cidx_v, sidx_v, rows_v, acc_sh, sem, semi):
        c = lax.axis_index("c")
        s = lax.axis_index("s")
        w = c * NS + s

        vz = rows_v.at[0]
        _fill2d(vz, CHUNK, D, 0.0)
        for j in range(RPS // CHUNK):
            pltpu.sync_copy(vz, acc_sh.at[pl.ds(s * RPS + j * CHUNK, CHUNK)])
        plsc.subcore_barrier()

        def load_half(h):
            sb = w * G + h * HALF
            db = NW * G + w * G + h * HALF

            def cb(j, _):
                gi = lax.iota(jnp.int32, L) + j * L
                val = jnp.where(gi < HALF, sb + gi, db + (gi - HALF))
                cidx_v[pl.ds(j * L, L)] = val
                return 0
            lax.fori_loop(0, 2 * HALF // L, cb, 0)
            pltpu.async_copy(e_hbm.at[cidx_v], sidx_v, semi).wait()

        def gather(g, b):
            pltpu.async_copy(t_hbm.at[sidx_v.at[g]], rows_v.at[b], sem.at[b])

        def gwait(b):
            pltpu.make_async_copy(
                t_hbm.at[sidx_v.at[0]], rows_v.at[b], sem.at[b]).wait()

        for h in range(G // HALF):
            load_half(h)
            gather(0, 0)

            def body(g, _):
                b = lax.rem(g, nbuf)
                gather(lax.rem(g + 1, HALF), lax.rem(g + 1, nbuf))
                gwait(b)
                pltpu.sync_copy(rows_v.at[b], acc_sh.at[sidx_v.at[HALF + g]],
                                add=True)
                return 0
            lax.fori_loop(0, HALF, body, 0)
            gwait(HALF % nbuf)

        plsc.subcore_barrier()
        for j in range(RPS // CHUNK):
            off = s * RPS + j * CHUNK
            b = rows_v.at[j % nbuf]
            pltpu.sync_copy(acc_sh.at[pl.ds(off, CHUNK)], b)
            pltpu.sync_copy(b, out_hbm.at[c].at[pl.ds(off, CHUNK)])

    return k(table, e2)


def _tc_norms_mm(dsrc, ddst, x_p, W1):
    def body(ds_ref, dd_ref, x_ref, w_ref, ns_ref, nd_ref, t_ref):
        dsv = ds_ref[...]
        nsv = jnp.where(dsv > 0, lax.rsqrt(jnp.maximum(dsv, 1.0)), 0.0)
        ddv = dd_ref[...]
        ndv = jnp.where(ddv > 0, lax.rsqrt(jnp.maximum(ddv, 1.0)), 0.0)
        ns_ref[...] = nsv
        nd_ref[...] = ndv
        t_ref[...] = jnp.dot(x_ref[...], w_ref[...],
                             preferred_element_type=jnp.float32) * nsv

    return pl.pallas_call(
        body,
        grid=(NPAD // BLK,),
        in_specs=[
            pl.BlockSpec((BLK, 1), lambda i: (i, 0)),
            pl.BlockSpec((BLK, 1), lambda i: (i, 0)),
            pl.BlockSpec((BLK, D), lambda i: (i, 0)),
            pl.BlockSpec((D, D), lambda i: (0, 0)),
        ],
        out_specs=[
            pl.BlockSpec((BLK, 1), lambda i: (i, 0)),
            pl.BlockSpec((BLK, 1), lambda i: (i, 0)),
            pl.BlockSpec((BLK, D), lambda i: (i, 0)),
        ],
        out_shape=[
            jax.ShapeDtypeStruct((NPAD, 1), jnp.float32),
            jax.ShapeDtypeStruct((NPAD, 1), jnp.float32),
            jax.ShapeDtypeStruct((NPAD, D), jnp.float32),
        ],
    )(dsrc, ddst, x_p, W1)


def _tc_boundary(p, nd, b, W, ns):
    def body(p_ref, nd_ref, b_ref, w_ref, ns_ref, h_ref, t_ref):
        agg = p_ref[0] + p_ref[1]
        h = jnp.maximum(agg * nd_ref[...] + b_ref[...], 0.0)
        h_ref[...] = h
        t_ref[...] = jnp.dot(h, w_ref[...],
                             preferred_element_type=jnp.float32) * ns_ref[...]

    return pl.pallas_call(
        body,
        grid=(NPAD // BLK,),
        in_specs=[
            pl.BlockSpec((NC, BLK, D), lambda i: (0, i, 0)),
            pl.BlockSpec((BLK, 1), lambda i: (i, 0)),
            pl.BlockSpec((1, D), lambda i: (0, 0)),
            pl.BlockSpec((D, D), lambda i: (0, 0)),
            pl.BlockSpec((BLK, 1), lambda i: (i, 0)),
        ],
        out_specs=[
            pl.BlockSpec((BLK, D), lambda i: (i, 0)),
            pl.BlockSpec((BLK, D), lambda i: (i, 0)),
        ],
        out_shape=[
            jax.ShapeDtypeStruct((NPAD, D), jnp.float32),
            jax.ShapeDtypeStruct((NPAD, D), jnp.float32),
        ],
    )(p, nd, b, W, ns)


def _tc_final(p, nd, b):
    def body(p_ref, nd_ref, b_ref, h_ref):
        agg = p_ref[0] + p_ref[1]
        h_ref[...] = agg * nd_ref[...] + b_ref[...]

    return pl.pallas_call(
        body,
        grid=(NPAD // BLK,),
        in_specs=[
            pl.BlockSpec((NC, BLK, D), lambda i: (0, i, 0)),
            pl.BlockSpec((BLK, 1), lambda i: (i, 0)),
            pl.BlockSpec((1, D), lambda i: (0, 0)),
        ],
        out_specs=pl.BlockSpec((BLK, D), lambda i: (i, 0)),
        out_shape=jax.ShapeDtypeStruct((NPAD, D), jnp.float32),
    )(p, nd, b)


def kernel(x, edge_index, W1, b1, W2, b2, W3, b3):
    n = x.shape[0]
    e = edge_index.shape[1]
    src = edge_index[0].astype(jnp.int32)
    dst = edge_index[1].astype(jnp.int32)
    padv = jnp.full((EPAD - e,), n, jnp.int32)
    src_p = jnp.concatenate([src, padv])
    dst_p = jnp.concatenate([dst, padv])
    e2 = jnp.concatenate([src_p.reshape(NW * G, CHUNK),
                          dst_p.reshape(NW * G, CHUNK)])
    x_p = jnp.concatenate([x, jnp.zeros((NPAD - n, D), jnp.float32)])

    deg2 = _sc_degree(e2)
    dsrc = deg2[0, :, 0:1]
    ddst = deg2[1, :, 0:1]
    ns_, nd_, t1 = _tc_norms_mm(dsrc, ddst, x_p, W1)
    p1 = _sc_edgepass(t1, e2)
    h1, t2 = _tc_boundary(p1, nd_, b1.reshape(1, D), W2, ns_)
    p2 = _sc_edgepass(t2, e2)
    h2, t3 = _tc_boundary(p2, nd_, b2.reshape(1, D), W3, ns_)
    p3 = _sc_edgepass(t3, e2)
    h3 = _tc_final(p3, nd_, b3.reshape(1, D))
    return (x, h1[:n], h2[:n], h3[:n])


# speedup vs baseline: 14.1240x; 2.7494x over previous
import functools

import jax
import jax.numpy as jnp
from jax import lax
from jax.experimental import pallas as pl
from jax.experimental.pallas import tpu as pltpu
from jax.experimental.pallas import tpu_sc as plsc

D = 128
NC, NS, L = 2, 16, 16
NW = NC * NS
CHUNK = 128
G = 80
HALF = 40
EPAD = NW * G * CHUNK
GD = EPAD // NS // CHUNK
NPAD = 10240
RPS = NPAD // NS
BLK = 512


def _mesh():
    return plsc.VectorSubcoreMesh(
        core_axis_name="c", subcore_axis_name="s",
        num_cores=NC, num_subcores=NS)


def _fill2d(ref, nrows, ncols, val):
    def rbody(i, _):
        def cbody(j, _):
            ref[i, pl.ds(j * L, L)] = jnp.full((L,), val, jnp.float32)
            return 0
        return lax.fori_loop(0, ncols // L, cbody, 0)
    lax.fori_loop(0, nrows, rbody, 0)


def _sc_degree(e2):
    @functools.partial(
        pl.kernel,
        out_type=jax.ShapeDtypeStruct((NC, NPAD, D), jnp.float32),
        mesh=_mesh(),
        scratch_types=[
            pltpu.VMEM((GD // 2,), jnp.int32),
            pltpu.VMEM((GD, CHUNK), jnp.int32),
            pltpu.VMEM((CHUNK, D), jnp.float32),
            pltpu.VMEM_SHARED((NPAD, D), jnp.float32),
            pltpu.SemaphoreType.DMA,
        ],
    )
    def k(e_hbm, out_hbm, cidx_v, idx_v, ones_v, acc_sh, semi):
        c = lax.axis_index("c")
        s = lax.axis_index("s")
        _fill2d(ones_v, CHUNK, D, 0.0)
        for j in range(RPS // CHUNK):
            pltpu.sync_copy(ones_v,
                            acc_sh.at[pl.ds(s * RPS + j * CHUNK, CHUNK)])
        hh = GD // 2
        base = c * NW * G + s * GD
        for h in range(2):
            def cb(j, _):
                cidx_v[pl.ds(j * L, L)] = lax.iota(jnp.int32, L) + (
                    base + h * hh + j * L)
                return 0
            lax.fori_loop(0, hh // L, cb, 0)
            pltpu.async_copy(e_hbm.at[cidx_v],
                             idx_v.at[pl.ds(h * hh, hh)], semi).wait()
        _fill2d(ones_v, CHUNK, D, 1.0)
        plsc.subcore_barrier()

        def body(g, _):
            pltpu.sync_copy(ones_v, acc_sh.at[idx_v.at[g]], add=True)
            return 0
        lax.fori_loop(0, GD, body, 0)
        plsc.subcore_barrier()
        for j in range(RPS // CHUNK):
            off = s * RPS + j * CHUNK
            pltpu.sync_copy(acc_sh.at[pl.ds(off, CHUNK)], ones_v)
            pltpu.sync_copy(ones_v, out_hbm.at[c].at[pl.ds(off, CHUNK)])

    return k(e2)


def _sc_edgepass(table, e2):
    nbuf = 2

    @functools.partial(
        pl.kernel,
        out_type=jax.ShapeDtypeStruct((NC, NPAD, D), jnp.float32),
        mesh=_mesh(),
        scratch_types=[
            pltpu.VMEM((2 * HALF,), jnp.int32),
            pltpu.VMEM((2 * HALF, CHUNK), jnp.int32),
            pltpu.VMEM((nbuf, CHUNK, D), jnp.float32),
            pltpu.VMEM_SHARED((NPAD, D), jnp.float32),
            pltpu.SemaphoreType.DMA((nbuf,)),
            pltpu.SemaphoreType.DMA,
        ],
    )
    def k(t_hbm, e_hbm, out_hbm, cidx_v, sidx_v, rows_v, acc_sh, sem, semi):
        c = lax.axis_index("c")
        s = lax.axis_index("s")
        w = c * NS + s

        vz = rows_v.at[0]
        _fill2d(vz, CHUNK, D, 0.0)
        for j in range(RPS // CHUNK):
            pltpu.sync_copy(vz, acc_sh.at[pl.ds(s * RPS + j * CHUNK, CHUNK)])
        plsc.subcore_barrier()

        def load_half(h):
            sb = w * G + h * HALF
            db = NW * G + w * G + h * HALF

            def cb(j, _):
                gi = lax.iota(jnp.int32, L) + j * L
                val = jnp.where(gi < HALF, sb + gi, db + (gi - HALF))
                cidx_v[pl.ds(j * L, L)] = val
                return 0
            lax.fori_loop(0, 2 * HALF // L, cb, 0)
            pltpu.async_copy(e_hbm.at[cidx_v], sidx_v, semi).wait()

        def gather(g, b):
            pltpu.async_copy(t_hbm.at[sidx_v.at[g]], rows_v.at[b], sem.at[b])

        def gwait(b):
            pltpu.make_async_copy(
                t_hbm.at[sidx_v.at[0]], rows_v.at[b], sem.at[b]).wait()

        for h in range(G // HALF):
            load_half(h)
            gather(0, 0)

            def body(g, _):
                b = lax.rem(g, nbuf)
                gather(lax.rem(g + 1, HALF), lax.rem(g + 1, nbuf))
                gwait(b)
                pltpu.sync_copy(rows_v.at[b], acc_sh.at[sidx_v.at[HALF + g]],
                                add=True)
                return 0
            lax.fori_loop(0, HALF, body, 0)
            gwait(HALF % nbuf)

        plsc.subcore_barrier()
        for j in range(RPS // CHUNK):
            off = s * RPS + j * CHUNK
            b = rows_v.at[j % nbuf]
            pltpu.sync_copy(acc_sh.at[pl.ds(off, CHUNK)], b)
            pltpu.sync_copy(b, out_hbm.at[c].at[pl.ds(off, CHUNK)])

    return k(table, e2)


def _tc_norms_mm(dsrc, ddst, x_p, W1):
    def body(ds_ref, dd_ref, x_ref, w_ref, ns_ref, nd_ref, t_ref):
        dsv = ds_ref[...]
        nsv = jnp.where(dsv > 0, lax.rsqrt(jnp.maximum(dsv, 1.0)), 0.0)
        ddv = dd_ref[...]
        ndv = jnp.where(ddv > 0, lax.rsqrt(jnp.maximum(ddv, 1.0)), 0.0)
        ns_ref[...] = nsv
        nd_ref[...] = ndv
        t_ref[...] = jnp.dot(x_ref[...], w_ref[...],
                             preferred_element_type=jnp.float32) * nsv

    return pl.pallas_call(
        body,
        grid=(NPAD // BLK,),
        in_specs=[
            pl.BlockSpec((BLK, 1), lambda i: (i, 0)),
            pl.BlockSpec((BLK, 1), lambda i: (i, 0)),
            pl.BlockSpec((BLK, D), lambda i: (i, 0)),
            pl.BlockSpec((D, D), lambda i: (0, 0)),
        ],
        out_specs=[
            pl.BlockSpec((BLK, 1), lambda i: (i, 0)),
            pl.BlockSpec((BLK, 1), lambda i: (i, 0)),
            pl.BlockSpec((BLK, D), lambda i: (i, 0)),
        ],
        out_shape=[
            jax.ShapeDtypeStruct((NPAD, 1), jnp.float32),
            jax.ShapeDtypeStruct((NPAD, 1), jnp.float32),
            jax.ShapeDtypeStruct((NPAD, D), jnp.float32),
        ],
    )(dsrc, ddst, x_p, W1)


def _tc_boundary(p, nd, b, W, ns):
    def body(p_ref, nd_ref, b_ref, w_ref, ns_ref, h_ref, t_ref):
        agg = p_ref[0] + p_ref[1]
        h = jnp.maximum(agg * nd_ref[...] + b_ref[...], 0.0)
        h_ref[...] = h
        t_ref[...] = jnp.dot(h, w_ref[...],
                             preferred_element_type=jnp.float32) * ns_ref[...]

    return pl.pallas_call(
        body,
        grid=(NPAD // BLK,),
        in_specs=[
            pl.BlockSpec((NC, BLK, D), lambda i: (0, i, 0)),
            pl.BlockSpec((BLK, 1), lambda i: (i, 0)),
            pl.BlockSpec((1, D), lambda i: (0, 0)),
            pl.BlockSpec((D, D), lambda i: (0, 0)),
            pl.BlockSpec((BLK, 1), lambda i: (i, 0)),
        ],
        out_specs=[
            pl.BlockSpec((BLK, D), lambda i: (i, 0)),
            pl.BlockSpec((BLK, D), lambda i: (i, 0)),
        ],
        out_shape=[
            jax.ShapeDtypeStruct((NPAD, D), jnp.float32),
            jax.ShapeDtypeStruct((NPAD, D), jnp.float32),
        ],
    )(p, nd, b, W, ns)


def _tc_final(p, nd, b):
    def body(p_ref, nd_ref, b_ref, h_ref):
        agg = p_ref[0] + p_ref[1]
        h_ref[...] = agg * nd_ref[...] + b_ref[...]

    return pl.pallas_call(
        body,
        grid=(NPAD // BLK,),
        in_specs=[
            pl.BlockSpec((NC, BLK, D), lambda i: (0, i, 0)),
            pl.BlockSpec((BLK, 1), lambda i: (i, 0)),
            pl.BlockSpec((1, D), lambda i: (0, 0)),
        ],
        out_specs=pl.BlockSpec((BLK, D), lambda i: (i, 0)),
        out_shape=jax.ShapeDtypeStruct((NPAD, D), jnp.float32),
    )(p, nd, b)


def kernel(x, edge_index, W1, b1, W2, b2, W3, b3):
    n = x.shape[0]
    e = edge_index.shape[1]
    src = edge_index[0].astype(jnp.int32)
    dst = edge_index[1].astype(jnp.int32)
    padv = n + (jnp.arange(EPAD - e, dtype=jnp.int32) % (NPAD - n))
    src_p = jnp.concatenate([src, padv])
    dst_p = jnp.concatenate([dst, padv])
    e2 = jnp.concatenate([src_p.reshape(NW * G, CHUNK),
                          dst_p.reshape(NW * G, CHUNK)])
    x_p = jnp.concatenate([x, jnp.zeros((NPAD - n, D), jnp.float32)])

    deg2 = _sc_degree(e2)
    dsrc = deg2[0, :, 0:1]
    ddst = deg2[1, :, 0:1]
    ns_, nd_, t1 = _tc_norms_mm(dsrc, ddst, x_p, W1)
    p1 = _sc_edgepass(t1, e2)
    h1, t2 = _tc_boundary(p1, nd_, b1.reshape(1, D), W2, ns_)
    p2 = _sc_edgepass(t2, e2)
    h2, t3 = _tc_boundary(p2, nd_, b2.reshape(1, D), W3, ns_)
    p3 = _sc_edgepass(t3, e2)
    h3 = _tc_final(p3, nd_, b3.reshape(1, D))
    return (x, h1[:n], h2[:n], h3[:n])
